# baseline (speedup 1.0000x reference)
import functools

import jax
import jax.numpy as jnp
from jax import lax
from jax.experimental import pallas as pl
from jax.experimental.pallas import tpu as pltpu
from jax.experimental.pallas import tpu_sc as plsc

_NUM_PT = 32
_NUM_T = 100000
_BATCH = 1024
_SEQ = 200
_TOK = _BATCH * _SEQ
_CHUNK = 4096
_NCHUNK = _TOK // _CHUNK
_NVEC = _CHUNK // 16
_VPU_TOK = _CHUNK // 2

_mesh = plsc.VectorSubcoreMesh(core_axis_name="c", subcore_axis_name="s")


@functools.partial(
    pl.kernel,
    mesh=_mesh,
    compiler_params=pltpu.CompilerParams(use_tc_tiling_on_sc=False,
                                         needs_layout_passes=False),
    out_type=jax.ShapeDtypeStruct((_NUM_PT, _TOK), jnp.float32),
    scratch_types=[
        pltpu.VMEM((_NUM_T,), jnp.float32),
        pltpu.VMEM((_CHUNK,), jnp.int32),
        pltpu.VMEM((_CHUNK,), jnp.int32),
        pltpu.VMEM((_CHUNK,), jnp.float32),
        pltpu.VMEM((_CHUNK,), jnp.float32),
        pltpu.SemaphoreType.DMA,
        pltpu.SemaphoreType.DMA,
        pltpu.SemaphoreType.DMA,
        pltpu.SemaphoreType.DMA,
    ],
)
def _sc_lookup(idx_hbm, rules_hbm, out_hbm, row_v, idx0_v, idx1_v,
               out0_v, out1_v, sem_in0, sem_in1, sem_out0, sem_out1):
    wid = lax.axis_index("s") * 2 + lax.axis_index("c")
    idx_b = (idx0_v, idx1_v)
    out_b = (out0_v, out1_v)
    sem_in = (sem_in0, sem_in1)
    sem_out = (sem_out0, sem_out1)

    def in_copy(ci, b):
        return pltpu.make_async_copy(idx_hbm.at[pl.ds(ci * _CHUNK, _CHUNK)],
                                     idx_b[b], sem_in[b])

    def out_copy(ci, b):
        return pltpu.make_async_copy(out_b[b],
                                     out_hbm.at[wid, pl.ds(ci * _CHUNK, _CHUNK)],
                                     sem_out[b])

    in_copy(0, 0).start()
    in_copy(1, 1).start()
    pltpu.sync_copy(rules_hbm.at[wid], row_v)

    def pair_body(p, _):
        for b in range(2):
            ci = p * 2 + b
            in_copy(ci, b).wait()

            @pl.when(p > 0)
            def _wait_out():
                out_copy(ci - 2, b).wait()

            iv, ov = idx_b[b], out_b[b]

            @plsc.parallel_loop(0, _CHUNK, step=16, unroll=8)
            def vec_body(t):
                ov[pl.ds(t, 16)] = plsc.load_gather(row_v, [iv[pl.ds(t, 16)]])

            out_copy(ci, b).start()

            @pl.when(ci + 2 < _NCHUNK)
            def _prefetch():
                in_copy(ci + 2, b).start()
        return 0

    lax.fori_loop(0, _NCHUNK // 2, pair_body, 0)
    for b in range(2):
        out_copy(_NCHUNK - 2 + b, b).wait()


def kernel(sentences, rules):
    idx5 = (sentences.astype(jnp.int32)
            .T.reshape(_SEQ // 8, 8, _BATCH // 128, 128)
            .transpose(0, 2, 1, 3)
            .reshape(_TOK))
    out5 = _sc_lookup(idx5, rules)
    return (out5
            .reshape(_NUM_PT, _SEQ // 8, _BATCH // 128, 8, 128)
            .transpose(2, 4, 0, 1, 3)
            .reshape(_BATCH, _NUM_PT, _SEQ))
